# baseline (speedup 1.0000x reference)
import functools

import jax
import jax.numpy as jnp
from jax import lax
from jax.experimental import pallas as pl
from jax.experimental.pallas import tpu as pltpu
from jax.experimental.pallas import tpu_sc as plsc

_BATCH = 16384
_HIST = 50
_D = 64
_V = 1000001
_VT = 7813
_PAIR_ROWS = _VT * 64

_info = plsc.get_sparse_core_info()
_NC, _NS = _info.num_cores, _info.num_subcores
_NW = _NC * _NS

_T1_STEPS = 248
_NBUF = 4


def _iota16():
    return jnp.arange(16, dtype=jnp.int32)


def _transpose_table(table_t):
    mesh = plsc.VectorSubcoreMesh(core_axis_name="c", subcore_axis_name="s")

    @functools.partial(
        pl.kernel,
        out_type=jax.ShapeDtypeStruct((_PAIR_ROWS, 128), jnp.float32),
        mesh=mesh,
        scratch_types=(
            [pltpu.VMEM((_D, 128), jnp.float32) for _ in range(_NBUF)]
            + [pltpu.VMEM((_D, 128), jnp.float32) for _ in range(_NBUF)]
            + [pltpu.SemaphoreType.DMA] * _NBUF
            + [pltpu.SemaphoreType.DMA] * _NBUF
        ),
        compiler_params=pltpu.CompilerParams(
            use_tc_tiling_on_sc=True, disable_bounds_checks=True,
            needs_layout_passes=False,
        ),
    )
    def k(src_hbm, dst_hbm, *scr):
        s_bufs = scr[:_NBUF]
        t_bufs = scr[_NBUF : 2 * _NBUF]
        rsem = scr[2 * _NBUF : 3 * _NBUF]
        wsem = scr[3 * _NBUF :]
        wid = lax.axis_index("s") * _NC + lax.axis_index("c")

        iota = _iota16()
        row_idx = [iota + ((l0 * 16) % _D) for l0 in range(8)]

        def read(i, b):
            c = wid + _NW * i

            @pl.when(c < _VT)
            def _():
                pltpu.async_copy(
                    src_hbm.at[:, pl.ds(c * 128, 128)], s_bufs[b], rsem[b]
                )

        def read_wait(b):
            pltpu.make_async_copy(
                src_hbm.at[:, pl.ds(0, 128)], s_bufs[b], rsem[b]
            ).wait()

        def write(i, b):
            c = wid + _NW * i
            pltpu.async_copy(
                t_bufs[b], dst_hbm.at[pl.ds(c * 64, 64)], wsem[b]
            )

        def write_wait(b):
            pltpu.make_async_copy(
                t_bufs[b], dst_hbm.at[pl.ds(0, 64)], wsem[b]
            ).wait()

        def shuffle(b):
            sb = s_bufs[b]
            tb = t_bufs[b]

            @pl.loop(0, 64)
            def _(q):
                for l0 in range(8):
                    col = jnp.full((16,), 2 * q + (l0 * 16) // _D, jnp.int32)
                    v = plsc.load_gather(sb, [row_idx[l0], col])
                    tb[q, pl.ds(l0 * 16, 16)] = v

        for g in range(2):
            read(g, g)

        @pl.loop(0, _T1_STEPS, step=_NBUF)
        def _(s):
            for b in range(_NBUF):
                i = s + b
                c = wid + _NW * i

                @pl.when(c < _VT)
                def _():
                    read_wait(b)

                    @pl.when(i >= _NBUF)
                    def _():
                        write_wait(b)

                    shuffle(b)
                    write(i, b)

                nb = (b + 2) % _NBUF
                read(i + 2, nb)

        for b in range(_NBUF):
            write_wait(b)

    return k(table_t)


def _gather(pairs, x_t):
    mesh = plsc.VectorSubcoreMesh(core_axis_name="c", subcore_axis_name="s")

    @functools.partial(
        pl.kernel,
        out_type=jax.ShapeDtypeStruct((_HIST, _D, _BATCH), jnp.float32),
        mesh=mesh,
        scratch_types=(
            [pltpu.VMEM((_HIST, 256), jnp.int32)]
            + [pltpu.VMEM((128, 128), jnp.float32) for _ in range(_NBUF)]
            + [pltpu.VMEM((_D, 128), jnp.float32) for _ in range(_NBUF)]
            + [pltpu.VMEM((128,), jnp.int32) for _ in range(_NBUF)]
            + [pltpu.VMEM((128,), jnp.int32) for _ in range(_NBUF)]
            + [pltpu.SemaphoreType.DMA] * _NBUF
            + [pltpu.SemaphoreType.DMA] * _NBUF
        ),
        compiler_params=pltpu.CompilerParams(
            use_tc_tiling_on_sc=True, needs_layout_passes=False
        ),
    )
    def k(pairs_hbm, x_hbm, out_hbm, idx_v, *scr):
        g_bufs = scr[:_NBUF]
        t_bufs = scr[_NBUF : 2 * _NBUF]
        p_bufs = scr[2 * _NBUF : 3 * _NBUF]
        a_bufs = scr[3 * _NBUF : 4 * _NBUF]
        gsem = scr[4 * _NBUF : 5 * _NBUF]
        wsem = scr[5 * _NBUF :]
        wid = lax.axis_index("s") * _NC + lax.axis_index("c")
        b0 = wid * (_BATCH // _NW)

        iota = _iota16()
        row_idx = [iota + l0 * 16 for l0 in range(8)]

        def prep_and_fire(g, b):
            h = g >> 1
            cl = g & 1
            for j in range(8):
                xv = idx_v[h, pl.ds(cl * 128 + j * 16, 16)]
                p_bufs[b][pl.ds(j * 16, 16)] = xv >> 1
                a_bufs[b][pl.ds(j * 16, 16)] = (xv & 1) * _D
            pltpu.async_copy(pairs_hbm.at[p_bufs[b]], g_bufs[b], gsem[b])

        def gather_wait(b):
            pltpu.make_async_copy(
                pairs_hbm.at[p_bufs[b]], g_bufs[b], gsem[b]
            ).wait()

        def write(g, half, b):
            h = g >> 1
            cg = b0 + half * 256 + (g & 1) * 128
            pltpu.async_copy(
                t_bufs[b], out_hbm.at[h, :, pl.ds(cg, 128)], wsem[b]
            )

        def write_wait(b):
            pltpu.make_async_copy(
                t_bufs[b], out_hbm.at[0, :, pl.ds(0, 128)], wsem[b]
            ).wait()

        def shuffle(b):
            gb = g_bufs[b]
            tb = t_bufs[b]
            par = [a_bufs[b][pl.ds(l0 * 16, 16)] for l0 in range(8)]

            @pl.loop(0, _D)
            def _(d):
                for l0 in range(8):
                    v = plsc.load_gather(gb, [row_idx[l0], par[l0] + d])
                    tb[d, pl.ds(l0 * 16, 16)] = v

        for half in range(2):
            pltpu.sync_copy(
                x_hbm.at[:, pl.ds(b0 + half * 256, 256)], idx_v
            )
            for g in range(2):
                prep_and_fire(g, g)

            @pl.loop(0, 100, step=_NBUF)
            def _(s):
                for b in range(_NBUF):
                    g = s + b
                    gather_wait(b)

                    if half == 0:
                        @pl.when(g >= _NBUF)
                        def _():
                            write_wait(b)
                    else:
                        write_wait(b)

                    shuffle(b)
                    write(g, half, b)
                    nb = (b + 2) % _NBUF
                    ng = g + 2

                    @pl.when(ng < 100)
                    def _():
                        prep_and_fire(ng, nb)

        for b in range(_NBUF):
            write_wait(b)

    return k(pairs, x_t)


def kernel(x, table):
    pairs = _transpose_table(table.T)
    out_p = _gather(pairs, x.astype(jnp.int32).T)
    return jnp.transpose(out_p, (2, 0, 1))


# speedup vs baseline: 1.8701x; 1.8701x over previous
import functools

import jax
import jax.numpy as jnp
from jax import lax
from jax.experimental import pallas as pl
from jax.experimental.pallas import tpu as pltpu
from jax.experimental.pallas import tpu_sc as plsc

_BATCH = 16384
_HIST = 50
_D = 64
_V = 1000001
_VT = 7813
_PAIR_ROWS = _VT * 64

_info = plsc.get_sparse_core_info()
_NC, _NS = _info.num_cores, _info.num_subcores
_NW = _NC * _NS

_T1_STEPS = 248
_NBUF = 4


def _iota16():
    return jnp.arange(16, dtype=jnp.int32)


def _transpose_table(table_t):
    mesh = plsc.VectorSubcoreMesh(core_axis_name="c", subcore_axis_name="s")

    @functools.partial(
        pl.kernel,
        out_type=jax.ShapeDtypeStruct((_PAIR_ROWS, 128), jnp.float32),
        mesh=mesh,
        scratch_types=(
            [pltpu.VMEM((_D, 128), jnp.float32) for _ in range(_NBUF)]
            + [pltpu.VMEM((_D, 128), jnp.float32) for _ in range(_NBUF)]
            + [pltpu.SemaphoreType.DMA] * _NBUF
            + [pltpu.SemaphoreType.DMA] * _NBUF
        ),
        compiler_params=pltpu.CompilerParams(
            use_tc_tiling_on_sc=True, disable_bounds_checks=True,
            needs_layout_passes=False,
        ),
    )
    def k(src_hbm, dst_hbm, *scr):
        s_bufs = scr[:_NBUF]
        t_bufs = scr[_NBUF : 2 * _NBUF]
        rsem = scr[2 * _NBUF : 3 * _NBUF]
        wsem = scr[3 * _NBUF :]
        wid = lax.axis_index("s") * _NC + lax.axis_index("c")

        iota = _iota16()
        row_idx = [iota + ((l0 * 16) % _D) for l0 in range(8)]

        def read(i, b):
            c = wid + _NW * i

            @pl.when(c < _VT)
            def _():
                pltpu.async_copy(
                    src_hbm.at[:, pl.ds(c * 128, 128)], s_bufs[b], rsem[b]
                )

        def read_wait(b):
            pltpu.make_async_copy(
                src_hbm.at[:, pl.ds(0, 128)], s_bufs[b], rsem[b]
            ).wait()

        def write(i, b):
            c = wid + _NW * i
            pltpu.async_copy(
                t_bufs[b], dst_hbm.at[pl.ds(c * 64, 64)], wsem[b]
            )

        def write_wait(b):
            pltpu.make_async_copy(
                t_bufs[b], dst_hbm.at[pl.ds(0, 64)], wsem[b]
            ).wait()

        def shuffle(b):
            sb = s_bufs[b]
            tb = t_bufs[b]

            @plsc.parallel_loop(0, 64, unroll=4)
            def _(q):
                for l0 in range(8):
                    col = jnp.full((16,), 2 * q + (l0 * 16) // _D, jnp.int32)
                    v = plsc.load_gather(sb, [row_idx[l0], col])
                    tb[q, pl.ds(l0 * 16, 16)] = v

        for g in range(2):
            read(g, g)

        @pl.loop(0, _T1_STEPS, step=_NBUF)
        def _(s):
            for b in range(_NBUF):
                i = s + b
                c = wid + _NW * i

                @pl.when(c < _VT)
                def _():
                    read_wait(b)

                    @pl.when(i >= _NBUF)
                    def _():
                        write_wait(b)

                    shuffle(b)
                    write(i, b)

                nb = (b + 2) % _NBUF
                read(i + 2, nb)

        for b in range(_NBUF):
            write_wait(b)

    return k(table_t)


def _gather(pairs, x_t):
    mesh = plsc.VectorSubcoreMesh(core_axis_name="c", subcore_axis_name="s")

    @functools.partial(
        pl.kernel,
        out_type=jax.ShapeDtypeStruct((_HIST, _D, _BATCH), jnp.float32),
        mesh=mesh,
        scratch_types=(
            [pltpu.VMEM((_HIST, 256), jnp.int32)]
            + [pltpu.VMEM((128, 128), jnp.float32) for _ in range(_NBUF)]
            + [pltpu.VMEM((_D, 128), jnp.float32) for _ in range(_NBUF)]
            + [pltpu.VMEM((128,), jnp.int32) for _ in range(_NBUF)]
            + [pltpu.VMEM((128,), jnp.int32) for _ in range(_NBUF)]
            + [pltpu.SemaphoreType.DMA] * _NBUF
            + [pltpu.SemaphoreType.DMA] * _NBUF
        ),
        compiler_params=pltpu.CompilerParams(
            use_tc_tiling_on_sc=True, needs_layout_passes=False
        ),
    )
    def k(pairs_hbm, x_hbm, out_hbm, idx_v, *scr):
        g_bufs = scr[:_NBUF]
        t_bufs = scr[_NBUF : 2 * _NBUF]
        p_bufs = scr[2 * _NBUF : 3 * _NBUF]
        a_bufs = scr[3 * _NBUF : 4 * _NBUF]
        gsem = scr[4 * _NBUF : 5 * _NBUF]
        wsem = scr[5 * _NBUF :]
        wid = lax.axis_index("s") * _NC + lax.axis_index("c")
        b0 = wid * (_BATCH // _NW)

        iota = _iota16()
        row_idx = [iota + l0 * 16 for l0 in range(8)]

        def prep_and_fire(g, b):
            h = g >> 1
            cl = g & 1
            for j in range(8):
                xv = idx_v[h, pl.ds(cl * 128 + j * 16, 16)]
                p_bufs[b][pl.ds(j * 16, 16)] = xv >> 1
                a_bufs[b][pl.ds(j * 16, 16)] = (xv & 1) * _D
            pltpu.async_copy(pairs_hbm.at[p_bufs[b]], g_bufs[b], gsem[b])

        def gather_wait(b):
            pltpu.make_async_copy(
                pairs_hbm.at[p_bufs[b]], g_bufs[b], gsem[b]
            ).wait()

        def write(g, half, b):
            h = g >> 1
            cg = b0 + half * 256 + (g & 1) * 128
            pltpu.async_copy(
                t_bufs[b], out_hbm.at[h, :, pl.ds(cg, 128)], wsem[b]
            )

        def write_wait(b):
            pltpu.make_async_copy(
                t_bufs[b], out_hbm.at[0, :, pl.ds(0, 128)], wsem[b]
            ).wait()

        def shuffle(b):
            gb = g_bufs[b]
            tb = t_bufs[b]
            par = [a_bufs[b][pl.ds(l0 * 16, 16)] for l0 in range(8)]

            @plsc.parallel_loop(0, _D, unroll=4)
            def _(d):
                for l0 in range(8):
                    v = plsc.load_gather(gb, [row_idx[l0], par[l0] + d])
                    tb[d, pl.ds(l0 * 16, 16)] = v

        for half in range(2):
            pltpu.sync_copy(
                x_hbm.at[:, pl.ds(b0 + half * 256, 256)], idx_v
            )
            for g in range(2):
                prep_and_fire(g, g)

            @pl.loop(0, 100, step=_NBUF)
            def _(s):
                for b in range(_NBUF):
                    g = s + b
                    gather_wait(b)

                    if half == 0:
                        @pl.when(g >= _NBUF)
                        def _():
                            write_wait(b)
                    else:
                        write_wait(b)

                    shuffle(b)
                    write(g, half, b)
                    nb = (b + 2) % _NBUF
                    ng = g + 2

                    @pl.when(ng < 100)
                    def _():
                        prep_and_fire(ng, nb)

        for b in range(_NBUF):
            write_wait(b)

    return k(pairs, x_t)


def kernel(x, table):
    pairs = _transpose_table(table.T)
    out_p = _gather(pairs, x.astype(jnp.int32).T)
    return jnp.transpose(out_p, (2, 0, 1))


# speedup vs baseline: 1.8704x; 1.0002x over previous
import functools

import jax
import jax.numpy as jnp
from jax import lax
from jax.experimental import pallas as pl
from jax.experimental.pallas import tpu as pltpu
from jax.experimental.pallas import tpu_sc as plsc

_BATCH = 16384
_HIST = 50
_D = 64
_V = 1000001
_VT = 7813
_PAIR_ROWS = _VT * 64

_info = plsc.get_sparse_core_info()
_NC, _NS = _info.num_cores, _info.num_subcores
_NW = _NC * _NS

_T1_STEPS = 248
_NBUF = 4


def _iota16():
    return jnp.arange(16, dtype=jnp.int32)


def _transpose_table(table_t):
    mesh = plsc.VectorSubcoreMesh(core_axis_name="c", subcore_axis_name="s")

    @functools.partial(
        pl.kernel,
        out_type=jax.ShapeDtypeStruct((_PAIR_ROWS, 128), jnp.float32),
        mesh=mesh,
        scratch_types=(
            [pltpu.VMEM((_D, 128), jnp.float32) for _ in range(_NBUF)]
            + [pltpu.VMEM((_D, 128), jnp.float32) for _ in range(_NBUF)]
            + [pltpu.SemaphoreType.DMA] * _NBUF
            + [pltpu.SemaphoreType.DMA] * _NBUF
        ),
        compiler_params=pltpu.CompilerParams(
            use_tc_tiling_on_sc=True, disable_bounds_checks=True,
            needs_layout_passes=False,
        ),
    )
    def k(src_hbm, dst_hbm, *scr):
        s_bufs = scr[:_NBUF]
        t_bufs = scr[_NBUF : 2 * _NBUF]
        rsem = scr[2 * _NBUF : 3 * _NBUF]
        wsem = scr[3 * _NBUF :]
        wid = lax.axis_index("s") * _NC + lax.axis_index("c")

        iota = _iota16()
        row_idx = [iota + ((l0 * 16) % _D) for l0 in range(8)]

        def read(i, b):
            c = wid + _NW * i

            @pl.when(c < _VT)
            def _():
                pltpu.async_copy(
                    src_hbm.at[:, pl.ds(c * 128, 128)], s_bufs[b], rsem[b]
                )

        def read_wait(b):
            pltpu.make_async_copy(
                src_hbm.at[:, pl.ds(0, 128)], s_bufs[b], rsem[b]
            ).wait()

        def write(i, b):
            c = wid + _NW * i
            pltpu.async_copy(
                t_bufs[b], dst_hbm.at[pl.ds(c * 64, 64)], wsem[b]
            )

        def write_wait(b):
            pltpu.make_async_copy(
                t_bufs[b], dst_hbm.at[pl.ds(0, 64)], wsem[b]
            ).wait()

        def shuffle(b):
            sb = s_bufs[b]
            tb = t_bufs[b]

            @plsc.parallel_loop(0, 64, unroll=8)
            def _(q):
                for l0 in range(8):
                    col = jnp.full((16,), 2 * q + (l0 * 16) // _D, jnp.int32)
                    v = plsc.load_gather(sb, [row_idx[l0], col])
                    tb[q, pl.ds(l0 * 16, 16)] = v

        for g in range(2):
            read(g, g)

        @pl.loop(0, _T1_STEPS, step=_NBUF)
        def _(s):
            for b in range(_NBUF):
                i = s + b
                c = wid + _NW * i

                @pl.when(c < _VT)
                def _():
                    read_wait(b)

                    @pl.when(i >= _NBUF)
                    def _():
                        write_wait(b)

                    shuffle(b)
                    write(i, b)

                nb = (b + 2) % _NBUF
                read(i + 2, nb)

        for b in range(_NBUF):
            write_wait(b)

    return k(table_t)


def _gather(pairs, x_t):
    mesh = plsc.VectorSubcoreMesh(core_axis_name="c", subcore_axis_name="s")

    @functools.partial(
        pl.kernel,
        out_type=jax.ShapeDtypeStruct((_HIST, _D, _BATCH), jnp.float32),
        mesh=mesh,
        scratch_types=(
            [pltpu.VMEM((_HIST, 256), jnp.int32)]
            + [pltpu.VMEM((128, 128), jnp.float32) for _ in range(_NBUF)]
            + [pltpu.VMEM((_D, 128), jnp.float32) for _ in range(_NBUF)]
            + [pltpu.VMEM((128,), jnp.int32) for _ in range(_NBUF)]
            + [pltpu.VMEM((128,), jnp.int32) for _ in range(_NBUF)]
            + [pltpu.SemaphoreType.DMA] * _NBUF
            + [pltpu.SemaphoreType.DMA] * _NBUF
        ),
        compiler_params=pltpu.CompilerParams(
            use_tc_tiling_on_sc=True, needs_layout_passes=False
        ),
    )
    def k(pairs_hbm, x_hbm, out_hbm, idx_v, *scr):
        g_bufs = scr[:_NBUF]
        t_bufs = scr[_NBUF : 2 * _NBUF]
        p_bufs = scr[2 * _NBUF : 3 * _NBUF]
        a_bufs = scr[3 * _NBUF : 4 * _NBUF]
        gsem = scr[4 * _NBUF : 5 * _NBUF]
        wsem = scr[5 * _NBUF :]
        wid = lax.axis_index("s") * _NC + lax.axis_index("c")
        b0 = wid * (_BATCH // _NW)

        iota = _iota16()
        row_idx = [iota + l0 * 16 for l0 in range(8)]

        def prep_and_fire(g, b):
            h = g >> 1
            cl = g & 1
            for j in range(8):
                xv = idx_v[h, pl.ds(cl * 128 + j * 16, 16)]
                p_bufs[b][pl.ds(j * 16, 16)] = xv >> 1
                a_bufs[b][pl.ds(j * 16, 16)] = (xv & 1) * _D
            pltpu.async_copy(pairs_hbm.at[p_bufs[b]], g_bufs[b], gsem[b])

        def gather_wait(b):
            pltpu.make_async_copy(
                pairs_hbm.at[p_bufs[b]], g_bufs[b], gsem[b]
            ).wait()

        def write(g, half, b):
            h = g >> 1
            cg = b0 + half * 256 + (g & 1) * 128
            pltpu.async_copy(
                t_bufs[b], out_hbm.at[h, :, pl.ds(cg, 128)], wsem[b]
            )

        def write_wait(b):
            pltpu.make_async_copy(
                t_bufs[b], out_hbm.at[0, :, pl.ds(0, 128)], wsem[b]
            ).wait()

        def shuffle(b):
            gb = g_bufs[b]
            tb = t_bufs[b]
            par = [a_bufs[b][pl.ds(l0 * 16, 16)] for l0 in range(8)]

            @plsc.parallel_loop(0, _D, unroll=8)
            def _(d):
                for l0 in range(8):
                    v = plsc.load_gather(gb, [row_idx[l0], par[l0] + d])
                    tb[d, pl.ds(l0 * 16, 16)] = v

        for half in range(2):
            pltpu.sync_copy(
                x_hbm.at[:, pl.ds(b0 + half * 256, 256)], idx_v
            )
            for g in range(2):
                prep_and_fire(g, g)

            @pl.loop(0, 100, step=_NBUF)
            def _(s):
                for b in range(_NBUF):
                    g = s + b
                    gather_wait(b)

                    if half == 0:
                        @pl.when(g >= _NBUF)
                        def _():
                            write_wait(b)
                    else:
                        write_wait(b)

                    shuffle(b)
                    write(g, half, b)
                    nb = (b + 2) % _NBUF
                    ng = g + 2

                    @pl.when(ng < 100)
                    def _():
                        prep_and_fire(ng, nb)

        for b in range(_NBUF):
            write_wait(b)

    return k(pairs, x_t)


def kernel(x, table):
    pairs = _transpose_table(table.T)
    out_p = _gather(pairs, x.astype(jnp.int32).T)
    return jnp.transpose(out_p, (2, 0, 1))


# speedup vs baseline: 2.1844x; 1.1679x over previous
import functools

import jax
import jax.numpy as jnp
from jax import lax
from jax.experimental import pallas as pl
from jax.experimental.pallas import tpu as pltpu
from jax.experimental.pallas import tpu_sc as plsc

_BATCH = 16384
_HIST = 50
_D = 64
_B = _BATCH * _HIST
_G = 128
_NUM_GROUPS = _B // _G

_info = plsc.get_sparse_core_info()
_NC, _NS = _info.num_cores, _info.num_subcores
_NW = _NC * _NS
_GPW = _NUM_GROUPS // _NW

_NBUF = 8
_LEAD = 4


def _embed_gather(table, idx2d):
    mesh = plsc.VectorSubcoreMesh(core_axis_name="c", subcore_axis_name="s")

    @functools.partial(
        pl.kernel,
        out_type=jax.ShapeDtypeStruct((_B, _D), jnp.float32),
        mesh=mesh,
        scratch_types=(
            [
                pltpu.VMEM((_GPW, _G), jnp.int32),
                pltpu.VMEM((_NBUF, _G, _D), jnp.float32),
            ]
            + [pltpu.SemaphoreType.DMA] * _NBUF
            + [pltpu.SemaphoreType.DMA] * _NBUF
        ),
        compiler_params=pltpu.CompilerParams(use_tc_tiling_on_sc=False),
    )
    def k(table_hbm, idx_hbm, out_hbm, idx_v, rows_v, *sems):
        gsem = sems[:_NBUF]
        ssem = sems[_NBUF:]
        wid = lax.axis_index("s") * _NC + lax.axis_index("c")
        g0 = wid * _GPW

        pltpu.sync_copy(idx_hbm.at[pl.ds(g0, _GPW)], idx_v)

        def gather(g, b):
            pltpu.async_copy(table_hbm.at[idx_v.at[g]], rows_v.at[b], gsem[b])

        def gather_wait(b):
            pltpu.make_async_copy(
                table_hbm.at[idx_v.at[0]], rows_v.at[b], gsem[b]
            ).wait()

        def scatter(g, b):
            pltpu.async_copy(
                rows_v.at[b], out_hbm.at[pl.ds((g0 + g) * _G, _G)], ssem[b]
            )

        def scatter_wait(b):
            pltpu.make_async_copy(
                rows_v.at[b], out_hbm.at[pl.ds(g0 * _G, _G)], ssem[b]
            ).wait()

        for b in range(_LEAD):
            gather(b, b)

        @pl.loop(0, _GPW, step=_NBUF)
        def _(s):
            for b in range(_NBUF):
                g = s + b
                gather_wait(b)
                scatter(g, b)
                nb = (b + _LEAD) % _NBUF
                ng = g + _LEAD

                @pl.when(ng < _GPW)
                def _():
                    @pl.when(ng >= _NBUF)
                    def _():
                        scatter_wait(nb)

                    gather(ng, nb)

        for b in range(_NBUF):
            scatter_wait(b)

    return k(table, idx2d)


def kernel(x, table):
    idx = x.reshape(_NUM_GROUPS, _G).astype(jnp.int32)
    out = _embed_gather(table, idx)
    return out.reshape(_BATCH, _HIST, _D)
